# baseline (speedup 1.0000x reference)
import functools

import jax
import jax.numpy as jnp
from jax import lax
from jax.experimental import pallas as pl
from jax.experimental.pallas import tpu as pltpu
from jax.experimental.pallas import tpu_sc as plsc

N = 10000
D = 128
E = 320000

NC = 2
NS = 16
NW = NC * NS
CH = 64
CPW = 160
IB = 20
NBUF = 4
GDEP = 3
NCHUNK = CPW * NW
EPAD = NCHUNK * CH
W0 = 320
W1 = 0
WB_TILES = 10
WB_ROWS = N // WB_TILES
ACC_ROWS = 10112
ZR = ACC_ROWS // NS

_mesh = plsc.VectorSubcoreMesh(core_axis_name="c", subcore_axis_name="s")


@functools.partial(
    pl.kernel,
    out_type=jax.ShapeDtypeStruct((NC, N, D), jnp.float32),
    mesh=_mesh,
    scratch_types=[
        pltpu.VMEM_SHARED((ACC_ROWS, D), jnp.float32),
        pltpu.VMEM((CPW, 1, CH), jnp.int32),
        pltpu.VMEM((CH, D), jnp.float32),
        pltpu.SemaphoreType.DMA,
    ],
)
def _deg_call(dst_hbm, ones_hbm, zeros_hbm, out_hbm, acc, didx, ones_v, sem):
    c = lax.axis_index("c")
    s = lax.axis_index("s")
    wid = s * NC + c
    pltpu.sync_copy(zeros_hbm, acc.at[pl.ds(s * ZR, ZR)])
    pltpu.sync_copy(ones_hbm, ones_v)
    pltpu.sync_copy(dst_hbm.at[pl.ds(wid * CPW, CPW)], didx)
    plsc.subcore_barrier()

    def fire(j, carry):
        pltpu.async_copy(ones_v, acc.at[didx.at[j, 0]], sem, add=True)
        return carry

    lax.fori_loop(0, CPW, fire, 0)

    def drain(j, carry):
        pltpu.make_async_copy(ones_v, acc.at[didx.at[0, 0]], sem).wait()
        return carry

    lax.fori_loop(0, CPW, drain, 0)
    plsc.subcore_barrier()

    @pl.when(s < WB_TILES)
    def _():
        pltpu.sync_copy(
            acc.at[pl.ds(s * WB_ROWS, WB_ROWS)],
            out_hbm.at[c, pl.ds(s * WB_ROWS, WB_ROWS)],
        )


@functools.partial(
    pl.kernel,
    out_type=jax.ShapeDtypeStruct((NC, N, D), jnp.float32),
    mesh=_mesh,
    scratch_types=[
        pltpu.VMEM_SHARED((ACC_ROWS, D), jnp.float32),
        pltpu.VMEM((2, IB, 1, CH), jnp.int32),
        pltpu.VMEM((2, IB, 1, CH), jnp.int32),
        pltpu.VMEM((NBUF, CH, D), jnp.float32),
        [pltpu.SemaphoreType.DMA] * NBUF,
        [pltpu.SemaphoreType.DMA] * NBUF,
    ],
)
def _agg_call(g_hbm, src_hbm, dst_hbm, zeros_hbm, out_hbm,
              acc, sidx, didx, rows, semg, sems):
    c = lax.axis_index("c")
    s = lax.axis_index("s")
    base = jnp.where(c == 0, s * W0, jnp.where(W1 > 0, NS * W0 + s * W1, 0))
    myw = jnp.where(c == 0, W0, W1)
    pltpu.sync_copy(zeros_hbm, acc.at[pl.ds(s * ZR, ZR)])
    pltpu.sync_copy(src_hbm.at[pl.ds(base, IB)], sidx.at[0])
    pltpu.sync_copy(dst_hbm.at[pl.ds(base, IB)], didx.at[0])
    plsc.subcore_barrier()

    @pl.when(myw > 0)
    def _():
        for t0 in range(GDEP):
            pltpu.async_copy(
                g_hbm.at[sidx.at[0, t0, 0]], rows.at[t0], semg[t0])

    def body(t, carry):
        k = lax.div(t, IB)
        slot = lax.rem(t, IB)
        kp = lax.rem(k, 2)
        for b in range(NBUF):
            @pl.when(lax.rem(t, NBUF) == b)
            def _():
                pltpu.make_async_copy(
                    g_hbm.at[sidx.at[0, 0, 0]], rows.at[b], semg[b]).wait()
                pltpu.async_copy(
                    rows.at[b], acc.at[didx.at[kp, slot, 0]], sems[b],
                    add=True)

                @pl.when(slot == IB - NBUF)
                def _():
                    @pl.when((k + 1) * IB < myw)
                    def _():
                        nk = lax.rem(k + 1, 2)
                        pltpu.sync_copy(
                            src_hbm.at[pl.ds(base + (k + 1) * IB, IB)],
                            sidx.at[nk])
                        pltpu.sync_copy(
                            dst_hbm.at[pl.ds(base + (k + 1) * IB, IB)],
                            didx.at[nk])

                @pl.when(t + GDEP < myw)
                def _():
                    nb = (b + GDEP) % NBUF

                    @pl.when(t + GDEP - NBUF >= 0)
                    def _():
                        pltpu.make_async_copy(
                            rows.at[nb], acc.at[didx.at[0, 0, 0]],
                            sems[nb]).wait()
                    nt = t + GDEP
                    nkp = lax.rem(lax.div(nt, IB), 2)
                    nslot = lax.rem(nt, IB)
                    pltpu.async_copy(
                        g_hbm.at[sidx.at[nkp, nslot, 0]], rows.at[nb],
                        semg[nb])
        return carry

    lax.fori_loop(0, myw, body, 0)

    @pl.when(myw > 0)
    def _():
        for b in range(NBUF):
            pltpu.make_async_copy(
                rows.at[b], acc.at[didx.at[0, 0, 0]], sems[b]).wait()
    plsc.subcore_barrier()

    @pl.when(s < WB_TILES)
    def _():
        pltpu.sync_copy(
            acc.at[pl.ds(s * WB_ROWS, WB_ROWS)],
            out_hbm.at[c, pl.ds(s * WB_ROWS, WB_ROWS)],
        )


_RB = 1000


def _dinv_from(deg_blk):
    deg = jnp.sum(deg_blk[...], axis=(0, 2)) + 1.0
    return lax.rsqrt(deg)[:, None]


def _tc_first_body(x_ref, w_ref, deg_ref, g_ref):
    dinv = _dinv_from(deg_ref)
    g_ref[...] = jnp.dot(x_ref[...], w_ref[...],
                         preferred_element_type=jnp.float32) * dinv


def _tc_mid_body(agg_ref, gp_ref, deg_ref, b_ref, w_ref, g_ref):
    dinv = _dinv_from(deg_ref)
    t = (jnp.sum(agg_ref[...], axis=0) + gp_ref[...]) * dinv + b_ref[...]
    t = jnp.maximum(t, 0.0)
    g_ref[...] = jnp.dot(t, w_ref[...],
                         preferred_element_type=jnp.float32) * dinv


def _tc_last_body(agg_ref, gp_ref, deg_ref, b_ref, out_ref):
    dinv = _dinv_from(deg_ref)
    out_ref[...] = (jnp.sum(agg_ref[...], axis=0) + gp_ref[...]) * dinv \
        + b_ref[...]


_spec_x = pl.BlockSpec((_RB, D), lambda i: (i, 0))
_spec_w = pl.BlockSpec((D, D), lambda i: (0, 0))
_spec_deg = pl.BlockSpec((NC, _RB, D), lambda i: (0, i, 0))
_spec_agg = pl.BlockSpec((NC, _RB, D), lambda i: (0, i, 0))
_spec_b = pl.BlockSpec((1, D), lambda i: (0, 0))

_tc_first = pl.pallas_call(
    _tc_first_body,
    grid=(N // _RB,),
    in_specs=[_spec_x, _spec_w, _spec_deg],
    out_specs=_spec_x,
    out_shape=jax.ShapeDtypeStruct((N, D), jnp.float32),
)

_tc_mid = pl.pallas_call(
    _tc_mid_body,
    grid=(N // _RB,),
    in_specs=[_spec_agg, _spec_x, _spec_deg, _spec_b, _spec_w],
    out_specs=_spec_x,
    out_shape=jax.ShapeDtypeStruct((N, D), jnp.float32),
)

_tc_last = pl.pallas_call(
    _tc_last_body,
    grid=(N // _RB,),
    in_specs=[_spec_agg, _spec_x, _spec_deg, _spec_b],
    out_specs=_spec_x,
    out_shape=jax.ShapeDtypeStruct((N, D), jnp.float32),
)


def kernel(x, edge_index, W1, b1, W2, b2, W3, b3):
    src = edge_index[0].astype(jnp.int32)
    dst = edge_index[1].astype(jnp.int32)
    npad = EPAD - E
    src_p = jnp.concatenate(
        [src, jnp.zeros((npad,), jnp.int32)]).reshape(NCHUNK, 1, CH)
    dst_p = jnp.concatenate(
        [dst, N + (jnp.arange(npad, dtype=jnp.int32) % (ACC_ROWS - N))]
    ).reshape(NCHUNK, 1, CH)

    ones1 = jnp.zeros((CH, D), jnp.float32).at[:, 0].set(1.0)
    zeros128 = jnp.zeros((ZR, D), jnp.float32)

    deg_parts = _deg_call(dst_p, ones1, zeros128)

    g1 = _tc_first(x, W1, deg_parts)
    agg1 = _agg_call(g1, src_p, dst_p, zeros128)
    g2 = _tc_mid(agg1, g1, deg_parts, b1.reshape(1, D), W2)
    agg2 = _agg_call(g2, src_p, dst_p, zeros128)
    g3 = _tc_mid(agg2, g2, deg_parts, b2.reshape(1, D), W3)
    agg3 = _agg_call(g3, src_p, dst_p, zeros128)
    out = _tc_last(agg3, g3, deg_parts, b3.reshape(1, D))
    return out


# speedup vs baseline: 3.7594x; 3.7594x over previous
import functools

import jax
import jax.numpy as jnp
from jax import lax
from jax.experimental import pallas as pl
from jax.experimental.pallas import tpu as pltpu
from jax.experimental.pallas import tpu_sc as plsc

N = 10000
D = 128
E = 320000

NC = 2
NS = 16
NW = NC * NS
CH = 64
IB = 20
NBUF = 4
GDEP = 3
NCHUNK = E // CH
CBASE = NCHUNK // NW
CEXTRA = NCHUNK - CBASE * NW
CMAX = CBASE + 1
NCPAD = ((NCHUNK + 7) // 8 + 1) * 8
WB_TILES = 10
WB_ROWS = N // WB_TILES
ACC_ROWS = N

_mesh = plsc.VectorSubcoreMesh(core_axis_name="c", subcore_axis_name="s")


@functools.partial(
    pl.kernel,
    out_type=jax.ShapeDtypeStruct((NC, N, D), jnp.float32),
    mesh=_mesh,
    scratch_types=[
        pltpu.VMEM_SHARED((ACC_ROWS, D), jnp.float32),
        pltpu.VMEM((CMAX, 1, CH), jnp.int32),
        pltpu.VMEM((CH, D), jnp.float32),
        pltpu.SemaphoreType.DMA,
    ],
)
def _deg_call(dst_hbm, ones_hbm, zeros_hbm, out_hbm, acc, didx, ones_v, sem):
    c = lax.axis_index("c")
    s = lax.axis_index("s")
    wid = s * NC + c
    base = CBASE * wid + jnp.minimum(wid, CEXTRA)
    myw = CBASE + jnp.where(wid < CEXTRA, 1, 0)

    @pl.when(s < WB_TILES)
    def _():
        pltpu.sync_copy(zeros_hbm, acc.at[pl.ds(s * WB_ROWS, WB_ROWS)])

    pltpu.sync_copy(ones_hbm, ones_v)
    pltpu.sync_copy(dst_hbm.at[pl.ds(base, CMAX)], didx)
    plsc.subcore_barrier()

    def fire(j, carry):
        pltpu.async_copy(ones_v, acc.at[didx.at[j, 0]], sem, add=True)
        return carry

    lax.fori_loop(0, myw, fire, 0)

    def drain(j, carry):
        pltpu.make_async_copy(ones_v, acc.at[didx.at[0, 0]], sem).wait()
        return carry

    lax.fori_loop(0, myw, drain, 0)
    plsc.subcore_barrier()

    @pl.when(s < WB_TILES)
    def _():
        pltpu.sync_copy(
            acc.at[pl.ds(s * WB_ROWS, WB_ROWS)],
            out_hbm.at[c, pl.ds(s * WB_ROWS, WB_ROWS)],
        )


@functools.partial(
    pl.kernel,
    out_type=jax.ShapeDtypeStruct((NC, N, D), jnp.float32),
    mesh=_mesh,
    scratch_types=[
        pltpu.VMEM_SHARED((ACC_ROWS, D), jnp.float32),
        pltpu.VMEM((2, IB, 1, CH), jnp.int32),
        pltpu.VMEM((2, IB, 1, CH), jnp.int32),
        pltpu.VMEM((NBUF, CH, D), jnp.float32),
        [pltpu.SemaphoreType.DMA] * NBUF,
        [pltpu.SemaphoreType.DMA] * NBUF,
    ],
)
def _agg_call(g_hbm, src_hbm, dst_hbm, zeros_hbm, out_hbm,
              acc, sidx, didx, rows, semg, sems):
    c = lax.axis_index("c")
    s = lax.axis_index("s")
    wid = s * NC + c
    base = CBASE * wid + jnp.minimum(wid, CEXTRA)
    myw = CBASE + jnp.where(wid < CEXTRA, 1, 0)

    @pl.when(s < WB_TILES)
    def _():
        pltpu.sync_copy(zeros_hbm, acc.at[pl.ds(s * WB_ROWS, WB_ROWS)])

    pltpu.sync_copy(src_hbm.at[pl.ds(base, IB)], sidx.at[0])
    pltpu.sync_copy(dst_hbm.at[pl.ds(base, IB)], didx.at[0])
    plsc.subcore_barrier()

    @pl.when(myw > 0)
    def _():
        for t0 in range(GDEP):
            pltpu.async_copy(
                g_hbm.at[sidx.at[0, t0, 0]], rows.at[t0], semg[t0])

    def body(t, carry):
        k = lax.div(t, IB)
        slot = lax.rem(t, IB)
        kp = lax.rem(k, 2)
        for b in range(NBUF):
            @pl.when(lax.rem(t, NBUF) == b)
            def _():
                pltpu.make_async_copy(
                    g_hbm.at[sidx.at[0, 0, 0]], rows.at[b], semg[b]).wait()
                pltpu.async_copy(
                    rows.at[b], acc.at[didx.at[kp, slot, 0]], sems[b],
                    add=True)

                @pl.when(slot == IB - NBUF)
                def _():
                    @pl.when((k + 1) * IB < myw)
                    def _():
                        nk = lax.rem(k + 1, 2)
                        pltpu.sync_copy(
                            src_hbm.at[pl.ds(base + (k + 1) * IB, IB)],
                            sidx.at[nk])
                        pltpu.sync_copy(
                            dst_hbm.at[pl.ds(base + (k + 1) * IB, IB)],
                            didx.at[nk])

                @pl.when(t + GDEP < myw)
                def _():
                    nb = (b + GDEP) % NBUF

                    @pl.when(t + GDEP - NBUF >= 0)
                    def _():
                        pltpu.make_async_copy(
                            rows.at[nb], acc.at[didx.at[0, 0, 0]],
                            sems[nb]).wait()
                    nt = t + GDEP
                    nkp = lax.rem(lax.div(nt, IB), 2)
                    nslot = lax.rem(nt, IB)
                    pltpu.async_copy(
                        g_hbm.at[sidx.at[nkp, nslot, 0]], rows.at[nb],
                        semg[nb])
        return carry

    lax.fori_loop(0, myw, body, 0)

    @pl.when(myw > 0)
    def _():
        for b in range(NBUF):
            pltpu.make_async_copy(
                rows.at[b], acc.at[didx.at[0, 0, 0]], sems[b]).wait()
    plsc.subcore_barrier()

    @pl.when(s < WB_TILES)
    def _():
        pltpu.sync_copy(
            acc.at[pl.ds(s * WB_ROWS, WB_ROWS)],
            out_hbm.at[c, pl.ds(s * WB_ROWS, WB_ROWS)],
        )


_RB = 1000


def _dinv_from(deg_blk):
    deg = jnp.sum(deg_blk[...], axis=(0, 2)) + 1.0
    return lax.rsqrt(deg)[:, None]


def _tc_first_body(x_ref, w_ref, deg_ref, g_ref):
    dinv = _dinv_from(deg_ref)
    g_ref[...] = jnp.dot(x_ref[...], w_ref[...],
                         preferred_element_type=jnp.float32) * dinv


def _tc_mid_body(agg_ref, gp_ref, deg_ref, b_ref, w_ref, g_ref):
    dinv = _dinv_from(deg_ref)
    t = (jnp.sum(agg_ref[...], axis=0) + gp_ref[...]) * dinv + b_ref[...]
    t = jnp.maximum(t, 0.0)
    g_ref[...] = jnp.dot(t, w_ref[...],
                         preferred_element_type=jnp.float32) * dinv


def _tc_last_body(agg_ref, gp_ref, deg_ref, b_ref, out_ref):
    dinv = _dinv_from(deg_ref)
    out_ref[...] = (jnp.sum(agg_ref[...], axis=0) + gp_ref[...]) * dinv \
        + b_ref[...]


_spec_x = pl.BlockSpec((_RB, D), lambda i: (i, 0))
_spec_w = pl.BlockSpec((D, D), lambda i: (0, 0))
_spec_deg = pl.BlockSpec((NC, _RB, D), lambda i: (0, i, 0))
_spec_agg = pl.BlockSpec((NC, _RB, D), lambda i: (0, i, 0))
_spec_b = pl.BlockSpec((1, D), lambda i: (0, 0))

_tc_first = pl.pallas_call(
    _tc_first_body,
    grid=(N // _RB,),
    in_specs=[_spec_x, _spec_w, _spec_deg],
    out_specs=_spec_x,
    out_shape=jax.ShapeDtypeStruct((N, D), jnp.float32),
)

_tc_mid = pl.pallas_call(
    _tc_mid_body,
    grid=(N // _RB,),
    in_specs=[_spec_agg, _spec_x, _spec_deg, _spec_b, _spec_w],
    out_specs=_spec_x,
    out_shape=jax.ShapeDtypeStruct((N, D), jnp.float32),
)

_tc_last = pl.pallas_call(
    _tc_last_body,
    grid=(N // _RB,),
    in_specs=[_spec_agg, _spec_x, _spec_deg, _spec_b],
    out_specs=_spec_x,
    out_shape=jax.ShapeDtypeStruct((N, D), jnp.float32),
)


def kernel(x, edge_index, W1, b1, W2, b2, W3, b3):
    src = edge_index[0].astype(jnp.int32)
    dst = edge_index[1].astype(jnp.int32)
    npad = NCPAD * CH - E
    src_p = jnp.concatenate(
        [src, jnp.zeros((npad,), jnp.int32)]).reshape(NCPAD, 1, CH)
    dst_p = jnp.concatenate(
        [dst, jnp.zeros((npad,), jnp.int32)]).reshape(NCPAD, 1, CH)

    ones1 = jnp.zeros((CH, D), jnp.float32).at[:, 0].set(1.0)
    zeros128 = jnp.zeros((WB_ROWS, D), jnp.float32)

    deg_parts = _deg_call(dst_p, ones1, zeros128)

    g1 = _tc_first(x, W1, deg_parts)
    agg1 = _agg_call(g1, src_p, dst_p, zeros128)
    g2 = _tc_mid(agg1, g1, deg_parts, b1.reshape(1, D), W2)
    agg2 = _agg_call(g2, src_p, dst_p, zeros128)
    g3 = _tc_mid(agg2, g2, deg_parts, b2.reshape(1, D), W3)
    agg3 = _agg_call(g3, src_p, dst_p, zeros128)
    out = _tc_last(agg3, g3, deg_parts, b3.reshape(1, D))
    return out


# speedup vs baseline: 3.8459x; 1.0230x over previous
import functools

import jax
import jax.numpy as jnp
from jax import lax
from jax.experimental import pallas as pl
from jax.experimental.pallas import tpu as pltpu
from jax.experimental.pallas import tpu_sc as plsc

N = 10000
D = 128
E = 320000

NC = 2
NS = 16
NW = NC * NS
CH = 80
IB = 25
NBUF = 3
GDEP = 2
NCHUNK = E // CH
CBASE = NCHUNK // NW
CEXTRA = NCHUNK - CBASE * NW
CMAX = CBASE + 1
NCPAD = ((NCHUNK + 7) // 8 + 1) * 8
WB_TILES = 10
WB_ROWS = N // WB_TILES
ACC_ROWS = N

_mesh = plsc.VectorSubcoreMesh(core_axis_name="c", subcore_axis_name="s")


@functools.partial(
    pl.kernel,
    out_type=jax.ShapeDtypeStruct((NC, N, D), jnp.float32),
    mesh=_mesh,
    scratch_types=[
        pltpu.VMEM_SHARED((ACC_ROWS, D), jnp.float32),
        pltpu.VMEM((CMAX, 1, CH), jnp.int32),
        pltpu.VMEM((CH, D), jnp.float32),
        pltpu.SemaphoreType.DMA,
    ],
)
def _deg_call(dst_hbm, ones_hbm, zeros_hbm, out_hbm, acc, didx, ones_v, sem):
    c = lax.axis_index("c")
    s = lax.axis_index("s")
    wid = s * NC + c
    base = CBASE * wid + jnp.minimum(wid, CEXTRA)
    myw = CBASE + jnp.where(wid < CEXTRA, 1, 0)

    @pl.when(s < WB_TILES)
    def _():
        pltpu.sync_copy(zeros_hbm, acc.at[pl.ds(s * WB_ROWS, WB_ROWS)])

    pltpu.sync_copy(ones_hbm, ones_v)
    pltpu.sync_copy(dst_hbm.at[pl.ds(base, CMAX)], didx)
    plsc.subcore_barrier()

    def fire(j, carry):
        pltpu.async_copy(ones_v, acc.at[didx.at[j, 0]], sem, add=True)
        return carry

    lax.fori_loop(0, myw, fire, 0)

    def drain(j, carry):
        pltpu.make_async_copy(ones_v, acc.at[didx.at[0, 0]], sem).wait()
        return carry

    lax.fori_loop(0, myw, drain, 0)
    plsc.subcore_barrier()

    @pl.when(s < WB_TILES)
    def _():
        pltpu.sync_copy(
            acc.at[pl.ds(s * WB_ROWS, WB_ROWS)],
            out_hbm.at[c, pl.ds(s * WB_ROWS, WB_ROWS)],
        )


@functools.partial(
    pl.kernel,
    out_type=jax.ShapeDtypeStruct((NC, N, D), jnp.float32),
    mesh=_mesh,
    scratch_types=[
        pltpu.VMEM_SHARED((ACC_ROWS, D), jnp.float32),
        pltpu.VMEM((2, IB, 1, CH), jnp.int32),
        pltpu.VMEM((2, IB, 1, CH), jnp.int32),
        pltpu.VMEM((NBUF, CH, D), jnp.float32),
        [pltpu.SemaphoreType.DMA] * NBUF,
        [pltpu.SemaphoreType.DMA] * NBUF,
    ],
)
def _agg_call(g_hbm, src_hbm, dst_hbm, zeros_hbm, out_hbm,
              acc, sidx, didx, rows, semg, sems):
    c = lax.axis_index("c")
    s = lax.axis_index("s")
    wid = s * NC + c
    base = CBASE * wid + jnp.minimum(wid, CEXTRA)
    myw = CBASE + jnp.where(wid < CEXTRA, 1, 0)

    @pl.when(s < WB_TILES)
    def _():
        pltpu.sync_copy(zeros_hbm, acc.at[pl.ds(s * WB_ROWS, WB_ROWS)])

    pltpu.sync_copy(src_hbm.at[pl.ds(base, IB)], sidx.at[0])
    pltpu.sync_copy(dst_hbm.at[pl.ds(base, IB)], didx.at[0])
    plsc.subcore_barrier()

    @pl.when(myw > 0)
    def _():
        for t0 in range(GDEP):
            pltpu.async_copy(
                g_hbm.at[sidx.at[0, t0, 0]], rows.at[t0], semg[t0])

    def body(t, carry):
        k = lax.div(t, IB)
        slot = lax.rem(t, IB)
        kp = lax.rem(k, 2)
        for b in range(NBUF):
            @pl.when(lax.rem(t, NBUF) == b)
            def _():
                pltpu.make_async_copy(
                    g_hbm.at[sidx.at[0, 0, 0]], rows.at[b], semg[b]).wait()
                pltpu.async_copy(
                    rows.at[b], acc.at[didx.at[kp, slot, 0]], sems[b],
                    add=True)

                @pl.when(slot == IB - NBUF)
                def _():
                    @pl.when((k + 1) * IB < myw)
                    def _():
                        nk = lax.rem(k + 1, 2)
                        pltpu.sync_copy(
                            src_hbm.at[pl.ds(base + (k + 1) * IB, IB)],
                            sidx.at[nk])
                        pltpu.sync_copy(
                            dst_hbm.at[pl.ds(base + (k + 1) * IB, IB)],
                            didx.at[nk])

                @pl.when(t + GDEP < myw)
                def _():
                    nb = (b + GDEP) % NBUF

                    @pl.when(t + GDEP - NBUF >= 0)
                    def _():
                        pltpu.make_async_copy(
                            rows.at[nb], acc.at[didx.at[0, 0, 0]],
                            sems[nb]).wait()
                    nt = t + GDEP
                    nkp = lax.rem(lax.div(nt, IB), 2)
                    nslot = lax.rem(nt, IB)
                    pltpu.async_copy(
                        g_hbm.at[sidx.at[nkp, nslot, 0]], rows.at[nb],
                        semg[nb])
        return carry

    lax.fori_loop(0, myw, body, 0)

    @pl.when(myw > 0)
    def _():
        for b in range(NBUF):
            pltpu.make_async_copy(
                rows.at[b], acc.at[didx.at[0, 0, 0]], sems[b]).wait()
    plsc.subcore_barrier()

    @pl.when(s < WB_TILES)
    def _():
        pltpu.sync_copy(
            acc.at[pl.ds(s * WB_ROWS, WB_ROWS)],
            out_hbm.at[c, pl.ds(s * WB_ROWS, WB_ROWS)],
        )


_RB = 1000


def _dinv_from(deg_blk):
    deg = jnp.sum(deg_blk[...], axis=(0, 2)) + 1.0
    return lax.rsqrt(deg)[:, None]


def _tc_first_body(x_ref, w_ref, deg_ref, g_ref):
    dinv = _dinv_from(deg_ref)
    g_ref[...] = jnp.dot(x_ref[...], w_ref[...],
                         preferred_element_type=jnp.float32) * dinv


def _tc_mid_body(agg_ref, gp_ref, deg_ref, b_ref, w_ref, g_ref):
    dinv = _dinv_from(deg_ref)
    t = (jnp.sum(agg_ref[...], axis=0) + gp_ref[...]) * dinv + b_ref[...]
    t = jnp.maximum(t, 0.0)
    g_ref[...] = jnp.dot(t, w_ref[...],
                         preferred_element_type=jnp.float32) * dinv


def _tc_last_body(agg_ref, gp_ref, deg_ref, b_ref, out_ref):
    dinv = _dinv_from(deg_ref)
    out_ref[...] = (jnp.sum(agg_ref[...], axis=0) + gp_ref[...]) * dinv \
        + b_ref[...]


_spec_x = pl.BlockSpec((_RB, D), lambda i: (i, 0))
_spec_w = pl.BlockSpec((D, D), lambda i: (0, 0))
_spec_deg = pl.BlockSpec((NC, _RB, D), lambda i: (0, i, 0))
_spec_agg = pl.BlockSpec((NC, _RB, D), lambda i: (0, i, 0))
_spec_b = pl.BlockSpec((1, D), lambda i: (0, 0))

_tc_first = pl.pallas_call(
    _tc_first_body,
    grid=(N // _RB,),
    in_specs=[_spec_x, _spec_w, _spec_deg],
    out_specs=_spec_x,
    out_shape=jax.ShapeDtypeStruct((N, D), jnp.float32),
)

_tc_mid = pl.pallas_call(
    _tc_mid_body,
    grid=(N // _RB,),
    in_specs=[_spec_agg, _spec_x, _spec_deg, _spec_b, _spec_w],
    out_specs=_spec_x,
    out_shape=jax.ShapeDtypeStruct((N, D), jnp.float32),
)

_tc_last = pl.pallas_call(
    _tc_last_body,
    grid=(N // _RB,),
    in_specs=[_spec_agg, _spec_x, _spec_deg, _spec_b],
    out_specs=_spec_x,
    out_shape=jax.ShapeDtypeStruct((N, D), jnp.float32),
)


def kernel(x, edge_index, W1, b1, W2, b2, W3, b3):
    src = edge_index[0].astype(jnp.int32)
    dst = edge_index[1].astype(jnp.int32)
    npad = NCPAD * CH - E
    src_p = jnp.concatenate(
        [src, jnp.zeros((npad,), jnp.int32)]).reshape(NCPAD, 1, CH)
    dst_p = jnp.concatenate(
        [dst, jnp.zeros((npad,), jnp.int32)]).reshape(NCPAD, 1, CH)

    ones1 = jnp.zeros((CH, D), jnp.float32).at[:, 0].set(1.0)
    zeros128 = jnp.zeros((WB_ROWS, D), jnp.float32)

    deg_parts = _deg_call(dst_p, ones1, zeros128)

    g1 = _tc_first(x, W1, deg_parts)
    agg1 = _agg_call(g1, src_p, dst_p, zeros128)
    g2 = _tc_mid(agg1, g1, deg_parts, b1.reshape(1, D), W2)
    agg2 = _agg_call(g2, src_p, dst_p, zeros128)
    g3 = _tc_mid(agg2, g2, deg_parts, b2.reshape(1, D), W3)
    agg3 = _agg_call(g3, src_p, dst_p, zeros128)
    out = _tc_last(agg3, g3, deg_parts, b3.reshape(1, D))
    return out


# speedup vs baseline: 3.8593x; 1.0035x over previous
import functools

import jax
import jax.numpy as jnp
from jax import lax
from jax.experimental import pallas as pl
from jax.experimental.pallas import tpu as pltpu
from jax.experimental.pallas import tpu_sc as plsc

N = 10000
D = 128
E = 320000

NC = 2
NS = 16
NW = NC * NS
CH = 80
IB = 25
NBUF = 3
GDEP = 2
NCHUNK = E // CH
CBASE = NCHUNK // NW
CEXTRA = NCHUNK - CBASE * NW
CMAX = CBASE + 1
NCPAD = ((NCHUNK + 7) // 8 + 1) * 8
WB_TILES = 10
WB_ROWS = N // WB_TILES
ACC_ROWS = N

_mesh = plsc.VectorSubcoreMesh(core_axis_name="c", subcore_axis_name="s")


@functools.partial(
    pl.kernel,
    out_type=jax.ShapeDtypeStruct((NC, N, D), jnp.float32),
    mesh=_mesh,
    scratch_types=[
        pltpu.VMEM_SHARED((ACC_ROWS, D), jnp.float32),
        pltpu.VMEM((CMAX, 1, CH), jnp.int32),
        pltpu.VMEM((CH, D), jnp.float32),
        pltpu.SemaphoreType.DMA,
    ],
)
def _deg_call(dst_hbm, ones_hbm, zeros_hbm, out_hbm, acc, didx, ones_v, sem):
    c = lax.axis_index("c")
    s = lax.axis_index("s")
    wid = s * NC + c
    base = CBASE * wid + jnp.minimum(wid, CEXTRA)
    myw = CBASE + jnp.where(wid < CEXTRA, 1, 0)

    @pl.when(s < WB_TILES)
    def _():
        pltpu.sync_copy(zeros_hbm, acc.at[pl.ds(s * WB_ROWS, WB_ROWS)])

    pltpu.sync_copy(ones_hbm, ones_v)
    pltpu.sync_copy(dst_hbm.at[pl.ds(base, CMAX)], didx)
    plsc.subcore_barrier()

    def fire(j, carry):
        pltpu.async_copy(ones_v, acc.at[didx.at[j, 0]], sem, add=True)
        return carry

    lax.fori_loop(0, myw, fire, 0)

    def drain(j, carry):
        pltpu.make_async_copy(ones_v, acc.at[didx.at[0, 0]], sem).wait()
        return carry

    lax.fori_loop(0, myw, drain, 0)
    plsc.subcore_barrier()

    @pl.when(s < WB_TILES)
    def _():
        pltpu.sync_copy(
            acc.at[pl.ds(s * WB_ROWS, WB_ROWS)],
            out_hbm.at[c, pl.ds(s * WB_ROWS, WB_ROWS)],
        )


@functools.partial(
    pl.kernel,
    out_type=jax.ShapeDtypeStruct((NC, N, D), jnp.float32),
    mesh=_mesh,
    scratch_types=[
        pltpu.VMEM_SHARED((ACC_ROWS, D), jnp.float32),
        pltpu.VMEM((2, IB, 1, CH), jnp.int32),
        pltpu.VMEM((2, IB, 1, CH), jnp.int32),
        pltpu.VMEM((NBUF, CH, D), jnp.float32),
        [pltpu.SemaphoreType.DMA] * NBUF,
        [pltpu.SemaphoreType.DMA] * NBUF,
    ],
)
def _agg_call(g_hbm, src_hbm, dst_hbm, zeros_hbm, out_hbm,
              acc, sidx, didx, rows, semg, sems):
    c = lax.axis_index("c")
    s = lax.axis_index("s")
    wid = s * NC + c
    base = CBASE * wid + jnp.minimum(wid, CEXTRA)
    myw = CBASE + jnp.where(wid < CEXTRA, 1, 0)

    @pl.when(s < WB_TILES)
    def _():
        pltpu.sync_copy(zeros_hbm, acc.at[pl.ds(s * WB_ROWS, WB_ROWS)])

    pltpu.sync_copy(src_hbm.at[pl.ds(base, IB)], sidx.at[0])
    pltpu.sync_copy(dst_hbm.at[pl.ds(base, IB)], didx.at[0])
    plsc.subcore_barrier()

    @pl.when(myw > 0)
    def _():
        for t0 in range(GDEP):
            pltpu.async_copy(
                g_hbm.at[sidx.at[0, t0, 0]], rows.at[t0], semg[t0])

    def body(t, carry):
        k = lax.div(t, IB)
        slot = lax.rem(t, IB)
        kp = lax.rem(k, 2)
        for b in range(NBUF):
            @pl.when(lax.rem(t, NBUF) == b)
            def _():
                pltpu.make_async_copy(
                    g_hbm.at[sidx.at[0, 0, 0]], rows.at[b], semg[b]).wait()
                pltpu.async_copy(
                    rows.at[b], acc.at[didx.at[kp, slot, 0]], sems[b],
                    add=True)

                @pl.when(slot == IB - NBUF)
                def _():
                    @pl.when((k + 1) * IB < myw)
                    def _():
                        nk = lax.rem(k + 1, 2)
                        pltpu.sync_copy(
                            src_hbm.at[pl.ds(base + (k + 1) * IB, IB)],
                            sidx.at[nk])
                        pltpu.sync_copy(
                            dst_hbm.at[pl.ds(base + (k + 1) * IB, IB)],
                            didx.at[nk])

                @pl.when(t + GDEP < myw)
                def _():
                    nb = (b + GDEP) % NBUF

                    @pl.when(t + GDEP - NBUF >= 0)
                    def _():
                        pltpu.make_async_copy(
                            rows.at[nb], acc.at[didx.at[0, 0, 0]],
                            sems[nb]).wait()
                    nt = t + GDEP
                    nkp = lax.rem(lax.div(nt, IB), 2)
                    nslot = lax.rem(nt, IB)
                    pltpu.async_copy(
                        g_hbm.at[sidx.at[nkp, nslot, 0]], rows.at[nb],
                        semg[nb])
        return carry

    lax.fori_loop(0, myw, body, 0)

    @pl.when(myw > 0)
    def _():
        for b in range(NBUF):
            pltpu.make_async_copy(
                rows.at[b], acc.at[didx.at[0, 0, 0]], sems[b]).wait()
    plsc.subcore_barrier()

    @pl.when(s < WB_TILES)
    def _():
        pltpu.sync_copy(
            acc.at[pl.ds(s * WB_ROWS, WB_ROWS)],
            out_hbm.at[c, pl.ds(s * WB_ROWS, WB_ROWS)],
        )


_RB = 1000


def _dinv_from(deg_blk):
    deg = jnp.sum(deg_blk[...], axis=(0, 2)) + 1.0
    return lax.rsqrt(deg)[:, None]


def _tc_first_body(x_ref, w_ref, deg_ref, g_ref, dinv_ref):
    dinv = _dinv_from(deg_ref)
    dinv_ref[...] = jnp.broadcast_to(dinv, (_RB, 8))
    g_ref[...] = jnp.dot(x_ref[...], w_ref[...],
                         preferred_element_type=jnp.float32) * dinv


def _tc_mid_body(agg_ref, gp_ref, dinv_ref, b_ref, w_ref, g_ref):
    dinv = dinv_ref[:, :1]
    t = (jnp.sum(agg_ref[...], axis=0) + gp_ref[...]) * dinv + b_ref[...]
    t = jnp.maximum(t, 0.0)
    g_ref[...] = jnp.dot(t, w_ref[...],
                         preferred_element_type=jnp.float32) * dinv


def _tc_last_body(agg_ref, gp_ref, dinv_ref, b_ref, out_ref):
    dinv = dinv_ref[:, :1]
    out_ref[...] = (jnp.sum(agg_ref[...], axis=0) + gp_ref[...]) * dinv \
        + b_ref[...]


_spec_x = pl.BlockSpec((_RB, D), lambda i: (i, 0))
_spec_w = pl.BlockSpec((D, D), lambda i: (0, 0))
_spec_deg = pl.BlockSpec((NC, _RB, D), lambda i: (0, i, 0))
_spec_agg = pl.BlockSpec((NC, _RB, D), lambda i: (0, i, 0))
_spec_b = pl.BlockSpec((1, D), lambda i: (0, 0))
_spec_dinv = pl.BlockSpec((_RB, 8), lambda i: (i, 0))

_tc_first = pl.pallas_call(
    _tc_first_body,
    grid=(N // _RB,),
    in_specs=[_spec_x, _spec_w, _spec_deg],
    out_specs=[_spec_x, _spec_dinv],
    out_shape=[jax.ShapeDtypeStruct((N, D), jnp.float32),
               jax.ShapeDtypeStruct((N, 8), jnp.float32)],
)

_tc_mid = pl.pallas_call(
    _tc_mid_body,
    grid=(N // _RB,),
    in_specs=[_spec_agg, _spec_x, _spec_dinv, _spec_b, _spec_w],
    out_specs=_spec_x,
    out_shape=jax.ShapeDtypeStruct((N, D), jnp.float32),
)

_tc_last = pl.pallas_call(
    _tc_last_body,
    grid=(N // _RB,),
    in_specs=[_spec_agg, _spec_x, _spec_dinv, _spec_b],
    out_specs=_spec_x,
    out_shape=jax.ShapeDtypeStruct((N, D), jnp.float32),
)


def kernel(x, edge_index, W1, b1, W2, b2, W3, b3):
    src = edge_index[0].astype(jnp.int32)
    dst = edge_index[1].astype(jnp.int32)
    npad = NCPAD * CH - E
    src_p = jnp.concatenate(
        [src, jnp.zeros((npad,), jnp.int32)]).reshape(NCPAD, 1, CH)
    dst_p = jnp.concatenate(
        [dst, jnp.zeros((npad,), jnp.int32)]).reshape(NCPAD, 1, CH)

    ones1 = jnp.zeros((CH, D), jnp.float32).at[:, 0].set(1.0)
    zeros128 = jnp.zeros((WB_ROWS, D), jnp.float32)

    deg_parts = _deg_call(dst_p, ones1, zeros128)

    g1, dinv8 = _tc_first(x, W1, deg_parts)
    agg1 = _agg_call(g1, src_p, dst_p, zeros128)
    g2 = _tc_mid(agg1, g1, dinv8, b1.reshape(1, D), W2)
    agg2 = _agg_call(g2, src_p, dst_p, zeros128)
    g3 = _tc_mid(agg2, g2, dinv8, b2.reshape(1, D), W3)
    agg3 = _agg_call(g3, src_p, dst_p, zeros128)
    out = _tc_last(agg3, g3, dinv8, b3.reshape(1, D))
    return out
